# Initial kernel scaffold; baseline (speedup 1.0000x reference)
import functools

import jax
import jax.numpy as jnp
from jax import lax
from jax.experimental import pallas as pl
from jax.experimental.pallas import tpu as pltpu
from jax.experimental.pallas import tpu_sc as plsc

NC = 2
NS = 16
LANES = 16
CHUNK = 80


def _make_sc_aggregate(n, h, r, e):
    ept_cnt = e // NS
    ept_agg = e // (NC * NS)
    rows_per_tile = n // NS
    c_per_tile = (r * n) // NS
    mesh = plsc.VectorSubcoreMesh(core_axis_name="c", subcore_axis_name="s")

    @functools.partial(
        pl.kernel,
        out_type=jax.ShapeDtypeStruct((NC * n, h), jnp.float32),
        mesh=mesh,
        scratch_types=[
            pltpu.VMEM_SHARED((n, h), jnp.float32),
            pltpu.VMEM_SHARED((r * n,), jnp.float32),
            pltpu.VMEM((CHUNK, h), jnp.float32),
            pltpu.VMEM((CHUNK,), jnp.int32),
            pltpu.VMEM((CHUNK,), jnp.int32),
            pltpu.VMEM((CHUNK,), jnp.int32),
            pltpu.VMEM((CHUNK,), jnp.int32),
            pltpu.VMEM((CHUNK,), jnp.int32),
            pltpu.VMEM((CHUNK,), jnp.float32),
            pltpu.VMEM((CHUNK,), jnp.float32),
        ],
    )
    def sc_aggregate(etype_hbm, src_hbm, dst_hbm, trans_hbm, zrows_hbm,
                     zflat_hbm, out_hbm, agg_sh, c_sh, rows_v, t_v, s_v,
                     d_v, g_v, rc_v, cv_v, ones_v):
        cid = lax.axis_index("c")
        sid = lax.axis_index("s")

        zr = sid * rows_per_tile
        pltpu.sync_copy(zrows_hbm.at[pl.ds(zr, rows_per_tile)],
                        agg_sh.at[pl.ds(zr, rows_per_tile)])
        zc = sid * c_per_tile
        pltpu.sync_copy(zflat_hbm.at[pl.ds(zc, c_per_tile)],
                        c_sh.at[pl.ds(zc, c_per_tile)])
        for j in range(CHUNK // LANES):
            ones_v[pl.ds(j * LANES, LANES)] = jnp.ones((LANES,), jnp.float32)
        plsc.subcore_barrier()

        cnt_base = sid * ept_cnt

        def count_step(k, _):
            e0 = cnt_base + k * CHUNK
            pltpu.sync_copy(etype_hbm.at[pl.ds(e0, CHUNK)], t_v)
            pltpu.sync_copy(dst_hbm.at[pl.ds(e0, CHUNK)], d_v)
            for j in range(CHUNK // LANES):
                sl = pl.ds(j * LANES, LANES)
                rc_v[sl] = t_v[sl] * n + d_v[sl]
            pltpu.sync_copy(ones_v, c_sh.at[rc_v], add=True)
            return ()

        lax.fori_loop(0, ept_cnt // CHUNK, count_step, ())
        plsc.subcore_barrier()

        agg_base = (cid * NS + sid) * ept_agg

        def agg_step(k, _):
            e0 = agg_base + k * CHUNK
            pltpu.sync_copy(etype_hbm.at[pl.ds(e0, CHUNK)], t_v)
            pltpu.sync_copy(src_hbm.at[pl.ds(e0, CHUNK)], s_v)
            pltpu.sync_copy(dst_hbm.at[pl.ds(e0, CHUNK)], d_v)
            for j in range(CHUNK // LANES):
                sl = pl.ds(j * LANES, LANES)
                t16 = t_v[sl]
                g_v[sl] = t16 * n + s_v[sl]
                rc_v[sl] = t16 * n + d_v[sl]
            pltpu.sync_copy(trans_hbm.at[g_v], rows_v)
            pltpu.sync_copy(c_sh.at[rc_v], cv_v)

            def scale_row(i, _):
                cw = plsc.load_gather(cv_v, [jnp.full((LANES,), i, jnp.int32)])
                w = 1.0 / jnp.maximum(cw, 1.0)
                for v in range(h // LANES):
                    sl = pl.ds(v * LANES, LANES)
                    rows_v[i, sl] = rows_v[i, sl] * w
                return ()

            lax.fori_loop(0, CHUNK, scale_row, ())
            pltpu.sync_copy(rows_v, agg_sh.at[d_v], add=True)
            return ()

        lax.fori_loop(0, ept_agg // CHUNK, agg_step, ())
        plsc.subcore_barrier()

        wr = sid * rows_per_tile
        pltpu.sync_copy(agg_sh.at[pl.ds(wr, rows_per_tile)],
                        out_hbm.at[pl.ds(cid * n + wr, rows_per_tile)])

    return sc_aggregate


def _tc_trans_body(x_ref, w_ref, o_ref):
    o_ref[0] = jnp.dot(x_ref[...], w_ref[0],
                       preferred_element_type=jnp.float32)


def _tc_head_body(p0_ref, p1_ref, xw_ref, resid_ref, muw_ref, mub_ref,
                  lvw_ref, lvb_ref, mu_ref, lv_ref):
    hid = p0_ref[...] + p1_ref[...] + xw_ref[0] + resid_ref[0, 0]
    hid = jnp.maximum(hid, 0.0)
    mu_ref[...] = jnp.dot(hid, muw_ref[...],
                          preferred_element_type=jnp.float32) + mub_ref[...]
    lv_ref[...] = jnp.dot(hid, lvw_ref[...],
                          preferred_element_type=jnp.float32) + lvb_ref[...]


def kernel(edge_index, edge_type, num_nodes, node_emb, W_rel, W_self,
           mu_W, mu_b, lv_W, lv_b):
    n, h = node_emb.shape
    r = W_rel.shape[0]
    e = edge_type.shape[0]
    lat = mu_W.shape[1]
    nb = 1000
    ngrid = n // nb

    src = edge_index[0]
    dst = edge_index[1]

    w_all = jnp.concatenate([W_rel, W_self[None]], axis=0)
    trans = pl.pallas_call(
        _tc_trans_body,
        grid=(r + 1, ngrid),
        in_specs=[
            pl.BlockSpec((nb, h), lambda i, j: (j, 0)),
            pl.BlockSpec((1, h, h), lambda i, j: (i, 0, 0)),
        ],
        out_specs=pl.BlockSpec((1, nb, h), lambda i, j: (i, j, 0)),
        out_shape=jax.ShapeDtypeStruct((r + 1, n, h), jnp.float32),
    )(node_emb, w_all)

    sc_fn = _make_sc_aggregate(n, h, r, e)
    partials = sc_fn(edge_type, src, dst, trans.reshape((r + 1) * n, h),
                     jnp.zeros((n, h), jnp.float32),
                     jnp.zeros((r * n,), jnp.float32))

    resid = (jnp.asarray(num_nodes) - n).astype(jnp.float32).reshape(1, 1)
    mu, lv = pl.pallas_call(
        _tc_head_body,
        grid=(ngrid,),
        in_specs=[
            pl.BlockSpec((nb, h), lambda j: (j, 0)),
            pl.BlockSpec((nb, h), lambda j: (ngrid + j, 0)),
            pl.BlockSpec((1, nb, h), lambda j: (r, j, 0)),
            pl.BlockSpec((1, 1), lambda j: (0, 0)),
            pl.BlockSpec((h, lat), lambda j: (0, 0)),
            pl.BlockSpec((1, lat), lambda j: (0, 0)),
            pl.BlockSpec((h, lat), lambda j: (0, 0)),
            pl.BlockSpec((1, lat), lambda j: (0, 0)),
        ],
        out_specs=[
            pl.BlockSpec((nb, lat), lambda j: (j, 0)),
            pl.BlockSpec((nb, lat), lambda j: (j, 0)),
        ],
        out_shape=[
            jax.ShapeDtypeStruct((n, lat), jnp.float32),
            jax.ShapeDtypeStruct((n, lat), jnp.float32),
        ],
    )(partials, partials, trans, resid, mu_W, mu_b.reshape(1, lat),
      lv_W, lv_b.reshape(1, lat))

    return (mu, mu, lv)


# speedup vs baseline: 10.2046x; 10.2046x over previous
import functools

import jax
import jax.numpy as jnp
from jax import lax
from jax.experimental import pallas as pl
from jax.experimental.pallas import tpu as pltpu
from jax.experimental.pallas import tpu_sc as plsc

NC = 2
NS = 16
LANES = 16
CHUNK = 80


def _make_sc_aggregate(n, h, r, e):
    ept_cnt = e // NS
    ept_agg = e // (NC * NS)
    rows_main = (n // NS) // 8 * 8
    rows_rem = n - NS * rows_main
    c_per_tile = (r * n) // NS
    c_zchunk = (c_per_tile + LANES - 1) // LANES * LANES
    mesh = plsc.VectorSubcoreMesh(core_axis_name="c", subcore_axis_name="s")

    @functools.partial(
        pl.kernel,
        out_type=jax.ShapeDtypeStruct((NC * n, h), jnp.float32),
        mesh=mesh,
        compiler_params=pltpu.CompilerParams(needs_layout_passes=False),
        scratch_types=[
            pltpu.VMEM_SHARED((n, h), jnp.float32),
            pltpu.VMEM_SHARED((r * n,), jnp.float32),
            pltpu.VMEM((CHUNK, h), jnp.float32),
            pltpu.VMEM((CHUNK,), jnp.int32),
            pltpu.VMEM((CHUNK,), jnp.int32),
            pltpu.VMEM((CHUNK,), jnp.int32),
            pltpu.VMEM((CHUNK,), jnp.int32),
            pltpu.VMEM((CHUNK,), jnp.int32),
            pltpu.VMEM((CHUNK,), jnp.float32),
            pltpu.VMEM((CHUNK,), jnp.float32),
            pltpu.VMEM((c_zchunk,), jnp.float32),
        ],
    )
    def sc_aggregate(etype_hbm, src_hbm, dst_hbm, trans_hbm, zrows_hbm,
                     out_hbm, agg_sh, c_sh, rows_v, t_v, s_v,
                     d_v, g_v, rc_v, cv_v, ones_v, zv_v):
        cid = lax.axis_index("c")
        sid = lax.axis_index("s")

        zr = sid * rows_main
        pltpu.sync_copy(zrows_hbm.at[pl.ds(zr, rows_main)],
                        agg_sh.at[pl.ds(zr, rows_main)])

        @pl.when(sid == 0)
        def _zero_tail():
            pltpu.sync_copy(zrows_hbm.at[pl.ds(NS * rows_main, rows_rem)],
                            agg_sh.at[pl.ds(NS * rows_main, rows_rem)])
        def zero_zv(i, _):
            zv_v[pl.ds(i * LANES, LANES)] = jnp.zeros((LANES,), jnp.float32)
            return ()

        lax.fori_loop(0, c_zchunk // LANES, zero_zv, ())
        zc = sid * c_per_tile
        pltpu.sync_copy(zv_v.at[pl.ds(0, c_per_tile)],
                        c_sh.at[pl.ds(zc, c_per_tile)])
        for j in range(CHUNK // LANES):
            ones_v[pl.ds(j * LANES, LANES)] = jnp.ones((LANES,), jnp.float32)
        plsc.subcore_barrier()

        cnt_base = sid * ept_cnt

        def count_step(k, _):
            e0 = cnt_base + k * CHUNK
            pltpu.sync_copy(etype_hbm.at[pl.ds(e0, CHUNK)], t_v)
            pltpu.sync_copy(dst_hbm.at[pl.ds(e0, CHUNK)], d_v)
            for j in range(CHUNK // LANES):
                sl = pl.ds(j * LANES, LANES)
                rc_v[sl] = t_v[sl] * n + d_v[sl]
            pltpu.sync_copy(ones_v, c_sh.at[rc_v], add=True)
            return ()

        lax.fori_loop(0, ept_cnt // CHUNK, count_step, ())
        plsc.subcore_barrier()

        agg_base = (cid * NS + sid) * ept_agg

        def agg_step(k, _):
            e0 = agg_base + k * CHUNK
            pltpu.sync_copy(etype_hbm.at[pl.ds(e0, CHUNK)], t_v)
            pltpu.sync_copy(src_hbm.at[pl.ds(e0, CHUNK)], s_v)
            pltpu.sync_copy(dst_hbm.at[pl.ds(e0, CHUNK)], d_v)
            for j in range(CHUNK // LANES):
                sl = pl.ds(j * LANES, LANES)
                t16 = t_v[sl]
                g_v[sl] = t16 * n + s_v[sl]
                rc_v[sl] = t16 * n + d_v[sl]
            pltpu.sync_copy(trans_hbm.at[g_v], rows_v)
            pltpu.sync_copy(c_sh.at[rc_v], cv_v)

            def scale_row(i, _):
                cw = plsc.load_gather(cv_v, [jnp.full((LANES,), i, jnp.int32)])
                w = 1.0 / jnp.maximum(cw, 1.0)
                for v in range(h // LANES):
                    sl = pl.ds(v * LANES, LANES)
                    rows_v[i, sl] = rows_v[i, sl] * w
                return ()

            lax.fori_loop(0, CHUNK, scale_row, ())
            pltpu.sync_copy(rows_v, agg_sh.at[d_v], add=True)
            return ()

        lax.fori_loop(0, ept_agg // CHUNK, agg_step, ())
        plsc.subcore_barrier()

        wr = sid * rows_main
        pltpu.sync_copy(agg_sh.at[pl.ds(wr, rows_main)],
                        out_hbm.at[pl.ds(cid * n + wr, rows_main)])

        @pl.when(sid == 0)
        def _write_tail():
            pltpu.sync_copy(agg_sh.at[pl.ds(NS * rows_main, rows_rem)],
                            out_hbm.at[pl.ds(cid * n + NS * rows_main,
                                             rows_rem)])

    return sc_aggregate


def _tc_trans_body(x_ref, w_ref, o_ref):
    o_ref[0] = jnp.dot(x_ref[...], w_ref[0],
                       preferred_element_type=jnp.float32)


def _tc_head_body(p0_ref, p1_ref, xw_ref, resid_ref, muw_ref, mub_ref,
                  lvw_ref, lvb_ref, mu_ref, lv_ref):
    hid = p0_ref[...] + p1_ref[...] + xw_ref[0] + resid_ref[0, 0]
    hid = jnp.maximum(hid, 0.0)
    mu_ref[...] = jnp.dot(hid, muw_ref[...],
                          preferred_element_type=jnp.float32) + mub_ref[...]
    lv_ref[...] = jnp.dot(hid, lvw_ref[...],
                          preferred_element_type=jnp.float32) + lvb_ref[...]


def kernel(edge_index, edge_type, num_nodes, node_emb, W_rel, W_self,
           mu_W, mu_b, lv_W, lv_b):
    n, h = node_emb.shape
    r = W_rel.shape[0]
    e = edge_type.shape[0]
    lat = mu_W.shape[1]
    nb = 1000
    ngrid = n // nb

    src = edge_index[0]
    dst = edge_index[1]

    w_all = jnp.concatenate([W_rel, W_self[None]], axis=0)
    trans = pl.pallas_call(
        _tc_trans_body,
        grid=(r + 1, ngrid),
        in_specs=[
            pl.BlockSpec((nb, h), lambda i, j: (j, 0)),
            pl.BlockSpec((1, h, h), lambda i, j: (i, 0, 0)),
        ],
        out_specs=pl.BlockSpec((1, nb, h), lambda i, j: (i, j, 0)),
        out_shape=jax.ShapeDtypeStruct((r + 1, n, h), jnp.float32),
    )(node_emb, w_all)

    sc_fn = _make_sc_aggregate(n, h, r, e)
    partials = sc_fn(edge_type, src, dst, trans.reshape((r + 1) * n, h),
                     jnp.zeros((n, h), jnp.float32))

    resid = (jnp.asarray(num_nodes) - n).astype(jnp.float32).reshape(1, 1)
    mu, lv = pl.pallas_call(
        _tc_head_body,
        grid=(ngrid,),
        in_specs=[
            pl.BlockSpec((nb, h), lambda j: (j, 0)),
            pl.BlockSpec((nb, h), lambda j: (ngrid + j, 0)),
            pl.BlockSpec((1, nb, h), lambda j: (r, j, 0)),
            pl.BlockSpec((1, 1), lambda j: (0, 0)),
            pl.BlockSpec((h, lat), lambda j: (0, 0)),
            pl.BlockSpec((1, lat), lambda j: (0, 0)),
            pl.BlockSpec((h, lat), lambda j: (0, 0)),
            pl.BlockSpec((1, lat), lambda j: (0, 0)),
        ],
        out_specs=[
            pl.BlockSpec((nb, lat), lambda j: (j, 0)),
            pl.BlockSpec((nb, lat), lambda j: (j, 0)),
        ],
        out_shape=[
            jax.ShapeDtypeStruct((n, lat), jnp.float32),
            jax.ShapeDtypeStruct((n, lat), jnp.float32),
        ],
    )(partials, partials, trans, resid, mu_W, mu_b.reshape(1, lat),
      lv_W, lv_b.reshape(1, lat))

    return (mu, mu, lv)


# speedup vs baseline: 14.7734x; 1.4477x over previous
import functools

import jax
import jax.numpy as jnp
from jax import lax
from jax.experimental import pallas as pl
from jax.experimental.pallas import tpu as pltpu
from jax.experimental.pallas import tpu_sc as plsc

NC = 2
NS = 16
LANES = 16
CHUNK = 80
NBUF = 5
GRP = CHUNK * NBUF


def _make_sc_aggregate(n, h, r, e):
    ept_cnt = e // NS
    ept_agg = e // (NC * NS)
    rows_main = (n // NS) // 8 * 8
    rows_rem = n - NS * rows_main
    c_per_tile = (r * n) // NS
    c_zchunk = (c_per_tile + LANES - 1) // LANES * LANES
    grp_cnt = ept_cnt // GRP
    grp_agg = ept_agg // GRP
    assert grp_cnt * GRP == ept_cnt
    assert grp_agg * GRP == ept_agg
    mesh = plsc.VectorSubcoreMesh(core_axis_name="c", subcore_axis_name="s")

    @functools.partial(
        pl.kernel,
        out_type=jax.ShapeDtypeStruct((NC * n, h), jnp.float32),
        mesh=mesh,
        compiler_params=pltpu.CompilerParams(needs_layout_passes=False),
        scratch_types=[
            pltpu.VMEM_SHARED((n, h), jnp.float32),
            pltpu.VMEM_SHARED((r * n,), jnp.float32),
            pltpu.VMEM((GRP,), jnp.int32),
            pltpu.VMEM((GRP,), jnp.int32),
            pltpu.VMEM((GRP,), jnp.int32),
            pltpu.VMEM((CHUNK,), jnp.float32),
            pltpu.VMEM((c_zchunk,), jnp.float32),
        ] + [pltpu.VMEM((CHUNK, h), jnp.float32),
             pltpu.VMEM((CHUNK,), jnp.int32),
             pltpu.VMEM((CHUNK,), jnp.int32),
             pltpu.VMEM((CHUNK,), jnp.int32),
             pltpu.VMEM((CHUNK,), jnp.float32)],
    )
    def sc_aggregate(etype_hbm, src_hbm, dst_hbm, trans_hbm, zrows_hbm,
                     out_hbm, agg_sh, c_sh, t_v, s_v, d_v, ones_v, zv_v,
                     rw_v, gx_v, rc_v, dd_v, cv_v):
        cid = lax.axis_index("c")
        sid = lax.axis_index("s")

        zr = sid * rows_main
        pltpu.sync_copy(zrows_hbm.at[pl.ds(zr, rows_main)],
                        agg_sh.at[pl.ds(zr, rows_main)])

        @pl.when(sid == 0)
        def _zero_tail():
            pltpu.sync_copy(zrows_hbm.at[pl.ds(NS * rows_main, rows_rem)],
                            agg_sh.at[pl.ds(NS * rows_main, rows_rem)])

        def zero_zv(i, _):
            zv_v[pl.ds(i * LANES, LANES)] = jnp.zeros((LANES,), jnp.float32)
            return ()

        lax.fori_loop(0, c_zchunk // LANES, zero_zv, ())
        zc = sid * c_per_tile
        pltpu.sync_copy(zv_v.at[pl.ds(0, c_per_tile)],
                        c_sh.at[pl.ds(zc, c_per_tile)])
        for j in range(CHUNK // LANES):
            ones_v[pl.ds(j * LANES, LANES)] = jnp.ones((LANES,), jnp.float32)
        plsc.subcore_barrier()

        cnt_base = sid * ept_cnt

        def count_group(g, _):
            e0 = cnt_base + g * GRP
            pltpu.sync_copy(etype_hbm.at[pl.ds(e0, GRP)], t_v)
            pltpu.sync_copy(dst_hbm.at[pl.ds(e0, GRP)], d_v)
            for b in range(NBUF):
                for j in range(CHUNK // LANES):
                    sl = pl.ds(b * CHUNK + j * LANES, LANES)
                    ob = pl.ds(j * LANES, LANES)
                    rc_v[ob] = t_v[sl] * n + d_v[sl]
                pltpu.sync_copy(ones_v, c_sh.at[rc_v], add=True)
            return ()

        lax.fori_loop(0, grp_cnt, count_group, ())
        plsc.subcore_barrier()

        agg_base = (cid * NS + sid) * ept_agg

        def agg_group(g, _):
            e0 = agg_base + g * GRP
            pltpu.sync_copy(etype_hbm.at[pl.ds(e0, GRP)], t_v)
            pltpu.sync_copy(src_hbm.at[pl.ds(e0, GRP)], s_v)
            pltpu.sync_copy(dst_hbm.at[pl.ds(e0, GRP)], d_v)
            for b in range(NBUF):
                for j in range(CHUNK // LANES):
                    sl = pl.ds(b * CHUNK + j * LANES, LANES)
                    ob = pl.ds(j * LANES, LANES)
                    t16 = t_v[sl]
                    d16 = d_v[sl]
                    gx_v[ob] = t16 * n + s_v[sl]
                    rc_v[ob] = t16 * n + d16
                    dd_v[ob] = d16
                pltpu.sync_copy(trans_hbm.at[gx_v], rw_v)
                pltpu.sync_copy(c_sh.at[rc_v], cv_v)

                def scale_row(i, _):
                    cw = plsc.load_gather(
                        cv_v, [jnp.full((LANES,), i, jnp.int32)])
                    w = 1.0 / jnp.maximum(cw, 1.0)
                    for v in range(h // LANES):
                        sl = pl.ds(v * LANES, LANES)
                        rw_v[i, sl] = rw_v[i, sl] * w
                    return ()

                lax.fori_loop(0, CHUNK, scale_row, ())
                pltpu.sync_copy(rw_v, agg_sh.at[dd_v], add=True)
            return ()

        lax.fori_loop(0, grp_agg, agg_group, ())
        plsc.subcore_barrier()

        wr = sid * rows_main
        pltpu.sync_copy(agg_sh.at[pl.ds(wr, rows_main)],
                        out_hbm.at[pl.ds(cid * n + wr, rows_main)])

        @pl.when(sid == 0)
        def _write_tail():
            pltpu.sync_copy(agg_sh.at[pl.ds(NS * rows_main, rows_rem)],
                            out_hbm.at[pl.ds(cid * n + NS * rows_main,
                                             rows_rem)])

    return sc_aggregate


def _tc_trans_body(x_ref, w_ref, o_ref):
    o_ref[0] = jnp.dot(x_ref[...], w_ref[0],
                       preferred_element_type=jnp.float32)


def _tc_head_body(p0_ref, p1_ref, xw_ref, resid_ref, muw_ref, mub_ref,
                  lvw_ref, lvb_ref, mu_ref, lv_ref):
    hid = p0_ref[...] + p1_ref[...] + xw_ref[0] + resid_ref[0, 0]
    hid = jnp.maximum(hid, 0.0)
    mu_ref[...] = jnp.dot(hid, muw_ref[...],
                          preferred_element_type=jnp.float32) + mub_ref[...]
    lv_ref[...] = jnp.dot(hid, lvw_ref[...],
                          preferred_element_type=jnp.float32) + lvb_ref[...]


def kernel(edge_index, edge_type, num_nodes, node_emb, W_rel, W_self,
           mu_W, mu_b, lv_W, lv_b):
    n, h = node_emb.shape
    r = W_rel.shape[0]
    e = edge_type.shape[0]
    lat = mu_W.shape[1]
    nb = 1000
    ngrid = n // nb

    src = edge_index[0]
    dst = edge_index[1]

    w_all = jnp.concatenate([W_rel, W_self[None]], axis=0)
    trans = pl.pallas_call(
        _tc_trans_body,
        grid=(r + 1, ngrid),
        in_specs=[
            pl.BlockSpec((nb, h), lambda i, j: (j, 0)),
            pl.BlockSpec((1, h, h), lambda i, j: (i, 0, 0)),
        ],
        out_specs=pl.BlockSpec((1, nb, h), lambda i, j: (i, j, 0)),
        out_shape=jax.ShapeDtypeStruct((r + 1, n, h), jnp.float32),
    )(node_emb, w_all)

    sc_fn = _make_sc_aggregate(n, h, r, e)
    partials = sc_fn(edge_type, src, dst, trans.reshape((r + 1) * n, h),
                     jnp.zeros((n, h), jnp.float32))

    resid = (jnp.asarray(num_nodes) - n).astype(jnp.float32).reshape(1, 1)
    mu, lv = pl.pallas_call(
        _tc_head_body,
        grid=(ngrid,),
        in_specs=[
            pl.BlockSpec((nb, h), lambda j: (j, 0)),
            pl.BlockSpec((nb, h), lambda j: (ngrid + j, 0)),
            pl.BlockSpec((1, nb, h), lambda j: (r, j, 0)),
            pl.BlockSpec((1, 1), lambda j: (0, 0)),
            pl.BlockSpec((h, lat), lambda j: (0, 0)),
            pl.BlockSpec((1, lat), lambda j: (0, 0)),
            pl.BlockSpec((h, lat), lambda j: (0, 0)),
            pl.BlockSpec((1, lat), lambda j: (0, 0)),
        ],
        out_specs=[
            pl.BlockSpec((nb, lat), lambda j: (j, 0)),
            pl.BlockSpec((nb, lat), lambda j: (j, 0)),
        ],
        out_shape=[
            jax.ShapeDtypeStruct((n, lat), jnp.float32),
            jax.ShapeDtypeStruct((n, lat), jnp.float32),
        ],
    )(partials, partials, trans, resid, mu_W, mu_b.reshape(1, lat),
      lv_W, lv_b.reshape(1, lat))

    return (mu, mu, lv)


# speedup vs baseline: 17.6485x; 1.1946x over previous
import functools

import jax
import jax.numpy as jnp
from jax import lax
from jax.experimental import pallas as pl
from jax.experimental.pallas import tpu as pltpu
from jax.experimental.pallas import tpu_sc as plsc

NC = 2
NS = 16
LANES = 16
CHUNK = 80
BCH = 128
SEG = 2000
NFULL = (SEG - CHUNK) // BCH


def _make_sc_aggregate(n, h, r, e):
    ept_cnt = e // NS
    ept_agg = e // (NC * NS)
    rows_main = (n // NS) // 8 * 8
    rows_rem = n - NS * rows_main
    c_per_tile = (r * n) // NS
    c_zchunk = (c_per_tile + LANES - 1) // LANES * LANES
    seg_cnt = ept_cnt // SEG
    seg_agg = ept_agg // SEG
    assert seg_cnt * SEG == ept_cnt and NFULL * BCH + CHUNK == SEG
    assert seg_agg * SEG == ept_agg
    mesh = plsc.VectorSubcoreMesh(core_axis_name="c", subcore_axis_name="s")

    @functools.partial(
        pl.kernel,
        out_type=jax.ShapeDtypeStruct((NC * n, h), jnp.float32),
        mesh=mesh,
        compiler_params=pltpu.CompilerParams(needs_layout_passes=False),
        scratch_types=[
            pltpu.VMEM_SHARED((n, h), jnp.float32),
            pltpu.VMEM_SHARED((r * n,), jnp.float32),
            pltpu.VMEM((SEG,), jnp.int32),
            pltpu.VMEM((SEG,), jnp.int32),
            pltpu.VMEM((SEG,), jnp.int32),
            pltpu.VMEM((BCH,), jnp.float32),
            pltpu.VMEM((c_zchunk,), jnp.float32),
        ] + [pltpu.VMEM((CHUNK, h), jnp.float32),
             pltpu.VMEM((CHUNK,), jnp.int32),
             pltpu.VMEM((CHUNK,), jnp.int32),
             pltpu.VMEM((CHUNK,), jnp.int32),
             pltpu.VMEM((CHUNK,), jnp.float32)]
          + [pltpu.VMEM((BCH, h), jnp.float32),
             pltpu.VMEM((BCH,), jnp.int32),
             pltpu.VMEM((BCH,), jnp.int32),
             pltpu.VMEM((BCH,), jnp.int32),
             pltpu.VMEM((BCH,), jnp.float32)],
    )
    def sc_aggregate(etype_hbm, src_hbm, dst_hbm, trans_hbm, zrows_hbm,
                     out_hbm, agg_sh, c_sh, t_v, s_v, d_v, ones_v, zv_v,
                     rw_v, gx_v, rc_v, dd_v, cv_v,
                     rw2_v, gx2_v, rc2_v, dd2_v, cv2_v):
        cid = lax.axis_index("c")
        sid = lax.axis_index("s")

        zr = sid * rows_main
        pltpu.sync_copy(zrows_hbm.at[pl.ds(zr, rows_main)],
                        agg_sh.at[pl.ds(zr, rows_main)])

        @pl.when(sid == 0)
        def _zero_tail():
            pltpu.sync_copy(zrows_hbm.at[pl.ds(NS * rows_main, rows_rem)],
                            agg_sh.at[pl.ds(NS * rows_main, rows_rem)])

        def zero_zv(i, _):
            zv_v[pl.ds(i * LANES, LANES)] = jnp.zeros((LANES,), jnp.float32)
            return ()

        lax.fori_loop(0, c_zchunk // LANES, zero_zv, ())
        zc = sid * c_per_tile
        pltpu.sync_copy(zv_v.at[pl.ds(0, c_per_tile)],
                        c_sh.at[pl.ds(zc, c_per_tile)])
        for j in range(BCH // LANES):
            ones_v[pl.ds(j * LANES, LANES)] = jnp.ones((LANES,), jnp.float32)
        plsc.subcore_barrier()

        cnt_base = sid * ept_cnt

        def count_seg(g, _):
            e0 = cnt_base + g * SEG
            pltpu.sync_copy(etype_hbm.at[pl.ds(e0, SEG)], t_v)
            pltpu.sync_copy(dst_hbm.at[pl.ds(e0, SEG)], d_v)

            def count_chunk(k, _):
                for j in range(BCH // LANES):
                    sl = pl.ds(k * BCH + j * LANES, LANES)
                    ob = pl.ds(j * LANES, LANES)
                    rc2_v[ob] = t_v[sl] * n + d_v[sl]
                pltpu.sync_copy(ones_v, c_sh.at[rc2_v], add=True)
                return ()

            lax.fori_loop(0, NFULL, count_chunk, ())
            for j in range(CHUNK // LANES):
                sl = pl.ds(NFULL * BCH + j * LANES, LANES)
                ob = pl.ds(j * LANES, LANES)
                rc_v[ob] = t_v[sl] * n + d_v[sl]
            pltpu.sync_copy(ones_v.at[pl.ds(0, CHUNK)], c_sh.at[rc_v],
                            add=True)
            return ()

        lax.fori_loop(0, seg_cnt, count_seg, ())
        plsc.subcore_barrier()

        agg_base = (cid * NS + sid) * ept_agg

        def agg_seg(g, _):
            e0 = agg_base + g * SEG
            pltpu.sync_copy(etype_hbm.at[pl.ds(e0, SEG)], t_v)
            pltpu.sync_copy(src_hbm.at[pl.ds(e0, SEG)], s_v)
            pltpu.sync_copy(dst_hbm.at[pl.ds(e0, SEG)], d_v)

            def do_chunk(base, sz, gxb, rcb, ddb, cvb, rwb):
                for j in range(sz // LANES):
                    sl = pl.ds(base + j * LANES, LANES)
                    ob = pl.ds(j * LANES, LANES)
                    t16 = t_v[sl]
                    d16 = d_v[sl]
                    gxb[ob] = t16 * n + s_v[sl]
                    rcb[ob] = t16 * n + d16
                    ddb[ob] = d16
                pltpu.sync_copy(trans_hbm.at[gxb], rwb)
                pltpu.sync_copy(c_sh.at[rcb], cvb)

                def scale_row(i, _):
                    cw = plsc.load_gather(
                        cvb, [jnp.full((LANES,), i, jnp.int32)])
                    w = 1.0 / jnp.maximum(cw, 1.0)
                    for v in range(h // LANES):
                        sl = pl.ds(v * LANES, LANES)
                        rwb[i, sl] = rwb[i, sl] * w
                    return ()

                lax.fori_loop(0, sz, scale_row, ())
                pltpu.sync_copy(rwb, agg_sh.at[ddb], add=True)

            def agg_chunk(k, _):
                do_chunk(k * BCH, BCH, gx2_v, rc2_v, dd2_v, cv2_v, rw2_v)
                return ()

            lax.fori_loop(0, NFULL, agg_chunk, ())
            do_chunk(NFULL * BCH, CHUNK, gx_v, rc_v, dd_v, cv_v, rw_v)
            return ()

        lax.fori_loop(0, seg_agg, agg_seg, ())
        plsc.subcore_barrier()

        wr = sid * rows_main
        pltpu.sync_copy(agg_sh.at[pl.ds(wr, rows_main)],
                        out_hbm.at[pl.ds(cid * n + wr, rows_main)])

        @pl.when(sid == 0)
        def _write_tail():
            pltpu.sync_copy(agg_sh.at[pl.ds(NS * rows_main, rows_rem)],
                            out_hbm.at[pl.ds(cid * n + NS * rows_main,
                                             rows_rem)])

    return sc_aggregate


def _tc_trans_body(x_ref, w_ref, o_ref):
    o_ref[0] = jnp.dot(x_ref[...], w_ref[0],
                       preferred_element_type=jnp.float32)


def _tc_head_body(p0_ref, p1_ref, xw_ref, resid_ref, muw_ref, mub_ref,
                  lvw_ref, lvb_ref, mu_ref, lv_ref):
    hid = p0_ref[...] + p1_ref[...] + xw_ref[0] + resid_ref[0, 0]
    hid = jnp.maximum(hid, 0.0)
    mu_ref[...] = jnp.dot(hid, muw_ref[...],
                          preferred_element_type=jnp.float32) + mub_ref[...]
    lv_ref[...] = jnp.dot(hid, lvw_ref[...],
                          preferred_element_type=jnp.float32) + lvb_ref[...]


def kernel(edge_index, edge_type, num_nodes, node_emb, W_rel, W_self,
           mu_W, mu_b, lv_W, lv_b):
    n, h = node_emb.shape
    r = W_rel.shape[0]
    e = edge_type.shape[0]
    lat = mu_W.shape[1]
    nb = 1000
    ngrid = n // nb

    src = edge_index[0]
    dst = edge_index[1]

    w_all = jnp.concatenate([W_rel, W_self[None]], axis=0)
    trans = pl.pallas_call(
        _tc_trans_body,
        grid=(r + 1, ngrid),
        in_specs=[
            pl.BlockSpec((nb, h), lambda i, j: (j, 0)),
            pl.BlockSpec((1, h, h), lambda i, j: (i, 0, 0)),
        ],
        out_specs=pl.BlockSpec((1, nb, h), lambda i, j: (i, j, 0)),
        out_shape=jax.ShapeDtypeStruct((r + 1, n, h), jnp.float32),
    )(node_emb, w_all)

    sc_fn = _make_sc_aggregate(n, h, r, e)
    partials = sc_fn(edge_type, src, dst, trans.reshape((r + 1) * n, h),
                     jnp.zeros((n, h), jnp.float32))

    resid = (jnp.asarray(num_nodes) - n).astype(jnp.float32).reshape(1, 1)
    mu, lv = pl.pallas_call(
        _tc_head_body,
        grid=(ngrid,),
        in_specs=[
            pl.BlockSpec((nb, h), lambda j: (j, 0)),
            pl.BlockSpec((nb, h), lambda j: (ngrid + j, 0)),
            pl.BlockSpec((1, nb, h), lambda j: (r, j, 0)),
            pl.BlockSpec((1, 1), lambda j: (0, 0)),
            pl.BlockSpec((h, lat), lambda j: (0, 0)),
            pl.BlockSpec((1, lat), lambda j: (0, 0)),
            pl.BlockSpec((h, lat), lambda j: (0, 0)),
            pl.BlockSpec((1, lat), lambda j: (0, 0)),
        ],
        out_specs=[
            pl.BlockSpec((nb, lat), lambda j: (j, 0)),
            pl.BlockSpec((nb, lat), lambda j: (j, 0)),
        ],
        out_shape=[
            jax.ShapeDtypeStruct((n, lat), jnp.float32),
            jax.ShapeDtypeStruct((n, lat), jnp.float32),
        ],
    )(partials, partials, trans, resid, mu_W, mu_b.reshape(1, lat),
      lv_W, lv_b.reshape(1, lat))

    return (mu, mu, lv)


# speedup vs baseline: 21.8503x; 1.2381x over previous
import functools

import jax
import jax.numpy as jnp
from jax import lax
from jax.experimental import pallas as pl
from jax.experimental.pallas import tpu as pltpu
from jax.experimental.pallas import tpu_sc as plsc

NC = 2
NS = 16
LANES = 16
CHUNK = 80
BCH = 128
SEG = 2000
NFULL = (SEG - CHUNK) // BCH


def _make_sc_aggregate(n, h, r, e):
    ept_cnt = e // NS
    ept_agg = e // (NC * NS)
    rows_main = (n // NS) // 8 * 8
    rows_rem = n - NS * rows_main
    c_per_tile = (r * n) // NS
    c_zchunk = (c_per_tile + LANES - 1) // LANES * LANES
    seg_cnt = ept_cnt // SEG
    seg_agg = ept_agg // SEG
    assert seg_cnt * SEG == ept_cnt and NFULL * BCH + CHUNK == SEG
    assert seg_agg * SEG == ept_agg
    mesh = plsc.VectorSubcoreMesh(core_axis_name="c", subcore_axis_name="s")

    @functools.partial(
        pl.kernel,
        out_type=jax.ShapeDtypeStruct((NC * n, h), jnp.float32),
        mesh=mesh,
        compiler_params=pltpu.CompilerParams(needs_layout_passes=False),
        scratch_types=[
            pltpu.VMEM_SHARED((n, h), jnp.float32),
            pltpu.VMEM_SHARED((r * n,), jnp.float32),
            pltpu.VMEM((SEG,), jnp.int32),
            pltpu.VMEM((SEG,), jnp.int32),
            pltpu.VMEM((SEG,), jnp.int32),
            pltpu.VMEM((BCH,), jnp.float32),
            pltpu.VMEM((c_zchunk,), jnp.float32),
        ] + [pltpu.VMEM((CHUNK, h), jnp.float32),
             pltpu.VMEM((CHUNK,), jnp.int32),
             pltpu.VMEM((CHUNK,), jnp.int32),
             pltpu.VMEM((CHUNK,), jnp.int32),
             pltpu.VMEM((CHUNK,), jnp.float32)]
          + [pltpu.VMEM((BCH, h), jnp.float32),
             pltpu.VMEM((BCH,), jnp.int32),
             pltpu.VMEM((BCH,), jnp.int32),
             pltpu.VMEM((BCH,), jnp.int32),
             pltpu.VMEM((BCH,), jnp.float32)],
    )
    def sc_aggregate(etype_hbm, src_hbm, dst_hbm, trans_hbm, zrows_hbm,
                     out_hbm, agg_sh, c_sh, t_v, s_v, d_v, ones_v, zv_v,
                     rw_v, gx_v, rc_v, dd_v, cv_v,
                     rw2_v, gx2_v, rc2_v, dd2_v, cv2_v):
        cid = lax.axis_index("c")
        sid = lax.axis_index("s")

        zr = sid * rows_main
        pltpu.sync_copy(zrows_hbm.at[pl.ds(zr, rows_main)],
                        agg_sh.at[pl.ds(zr, rows_main)])

        @pl.when(sid == 0)
        def _zero_tail():
            pltpu.sync_copy(zrows_hbm.at[pl.ds(NS * rows_main, rows_rem)],
                            agg_sh.at[pl.ds(NS * rows_main, rows_rem)])

        def zero_zv(i, _):
            zv_v[pl.ds(i * LANES, LANES)] = jnp.zeros((LANES,), jnp.float32)
            return ()

        lax.fori_loop(0, c_zchunk // LANES, zero_zv, ())
        zc = sid * c_per_tile
        pltpu.sync_copy(zv_v.at[pl.ds(0, c_per_tile)],
                        c_sh.at[pl.ds(zc, c_per_tile)])
        for j in range(BCH // LANES):
            ones_v[pl.ds(j * LANES, LANES)] = jnp.ones((LANES,), jnp.float32)
        plsc.subcore_barrier()

        cnt_base = sid * ept_cnt

        def count_seg(g, _):
            e0 = cnt_base + g * SEG
            pltpu.sync_copy(etype_hbm.at[pl.ds(e0, SEG)], t_v)
            pltpu.sync_copy(dst_hbm.at[pl.ds(e0, SEG)], d_v)

            def count_chunk(k, _):
                for j in range(BCH // LANES):
                    sl = pl.ds(k * BCH + j * LANES, LANES)
                    ob = pl.ds(j * LANES, LANES)
                    rc2_v[ob] = t_v[sl] * n + d_v[sl]
                pltpu.sync_copy(ones_v, c_sh.at[rc2_v], add=True)
                return ()

            lax.fori_loop(0, NFULL, count_chunk, ())
            for j in range(CHUNK // LANES):
                sl = pl.ds(NFULL * BCH + j * LANES, LANES)
                ob = pl.ds(j * LANES, LANES)
                rc_v[ob] = t_v[sl] * n + d_v[sl]
            pltpu.sync_copy(ones_v.at[pl.ds(0, CHUNK)], c_sh.at[rc_v],
                            add=True)
            return ()

        lax.fori_loop(0, seg_cnt, count_seg, ())
        plsc.subcore_barrier()

        agg_base = (cid * NS + sid) * ept_agg

        def agg_seg(g, _):
            e0 = agg_base + g * SEG
            pltpu.sync_copy(etype_hbm.at[pl.ds(e0, SEG)], t_v)
            pltpu.sync_copy(src_hbm.at[pl.ds(e0, SEG)], s_v)
            pltpu.sync_copy(dst_hbm.at[pl.ds(e0, SEG)], d_v)

            def do_chunk(base, sz, gxb, rcb, ddb, cvb, rwb):
                for j in range(sz // LANES):
                    sl = pl.ds(base + j * LANES, LANES)
                    ob = pl.ds(j * LANES, LANES)
                    t16 = t_v[sl]
                    d16 = d_v[sl]
                    gxb[ob] = t16 * n + s_v[sl]
                    rcb[ob] = t16 * n + d16
                    ddb[ob] = d16
                pltpu.sync_copy(trans_hbm.at[gxb], rwb)
                pltpu.sync_copy(c_sh.at[rcb], cvb)

                @plsc.parallel_loop(0, sz, 1, unroll=4)
                def scale_row(i):
                    cw = plsc.load_gather(
                        cvb, [jnp.full((LANES,), i, jnp.int32)])
                    w = 1.0 / jnp.maximum(cw, 1.0)
                    for v in range(h // LANES):
                        sl = pl.ds(v * LANES, LANES)
                        rwb[i, sl] = rwb[i, sl] * w
                pltpu.sync_copy(rwb, agg_sh.at[ddb], add=True)

            def agg_chunk(k, _):
                do_chunk(k * BCH, BCH, gx2_v, rc2_v, dd2_v, cv2_v, rw2_v)
                return ()

            lax.fori_loop(0, NFULL, agg_chunk, ())
            do_chunk(NFULL * BCH, CHUNK, gx_v, rc_v, dd_v, cv_v, rw_v)
            return ()

        lax.fori_loop(0, seg_agg, agg_seg, ())
        plsc.subcore_barrier()

        wr = sid * rows_main
        pltpu.sync_copy(agg_sh.at[pl.ds(wr, rows_main)],
                        out_hbm.at[pl.ds(cid * n + wr, rows_main)])

        @pl.when(sid == 0)
        def _write_tail():
            pltpu.sync_copy(agg_sh.at[pl.ds(NS * rows_main, rows_rem)],
                            out_hbm.at[pl.ds(cid * n + NS * rows_main,
                                             rows_rem)])

    return sc_aggregate


def _tc_trans_body(x_ref, w_ref, o_ref):
    o_ref[0] = jnp.dot(x_ref[...], w_ref[0],
                       preferred_element_type=jnp.float32)


def _tc_head_body(p0_ref, p1_ref, xw_ref, resid_ref, muw_ref, mub_ref,
                  lvw_ref, lvb_ref, mu_ref, lv_ref):
    hid = p0_ref[...] + p1_ref[...] + xw_ref[0] + resid_ref[0, 0]
    hid = jnp.maximum(hid, 0.0)
    mu_ref[...] = jnp.dot(hid, muw_ref[...],
                          preferred_element_type=jnp.float32) + mub_ref[...]
    lv_ref[...] = jnp.dot(hid, lvw_ref[...],
                          preferred_element_type=jnp.float32) + lvb_ref[...]


def kernel(edge_index, edge_type, num_nodes, node_emb, W_rel, W_self,
           mu_W, mu_b, lv_W, lv_b):
    n, h = node_emb.shape
    r = W_rel.shape[0]
    e = edge_type.shape[0]
    lat = mu_W.shape[1]
    nb = 1000
    ngrid = n // nb

    src = edge_index[0]
    dst = edge_index[1]

    w_all = jnp.concatenate([W_rel, W_self[None]], axis=0)
    trans = pl.pallas_call(
        _tc_trans_body,
        grid=(r + 1, ngrid),
        in_specs=[
            pl.BlockSpec((nb, h), lambda i, j: (j, 0)),
            pl.BlockSpec((1, h, h), lambda i, j: (i, 0, 0)),
        ],
        out_specs=pl.BlockSpec((1, nb, h), lambda i, j: (i, j, 0)),
        out_shape=jax.ShapeDtypeStruct((r + 1, n, h), jnp.float32),
    )(node_emb, w_all)

    sc_fn = _make_sc_aggregate(n, h, r, e)
    partials = sc_fn(edge_type, src, dst, trans.reshape((r + 1) * n, h),
                     jnp.zeros((n, h), jnp.float32))

    resid = (jnp.asarray(num_nodes) - n).astype(jnp.float32).reshape(1, 1)
    mu, lv = pl.pallas_call(
        _tc_head_body,
        grid=(ngrid,),
        in_specs=[
            pl.BlockSpec((nb, h), lambda j: (j, 0)),
            pl.BlockSpec((nb, h), lambda j: (ngrid + j, 0)),
            pl.BlockSpec((1, nb, h), lambda j: (r, j, 0)),
            pl.BlockSpec((1, 1), lambda j: (0, 0)),
            pl.BlockSpec((h, lat), lambda j: (0, 0)),
            pl.BlockSpec((1, lat), lambda j: (0, 0)),
            pl.BlockSpec((h, lat), lambda j: (0, 0)),
            pl.BlockSpec((1, lat), lambda j: (0, 0)),
        ],
        out_specs=[
            pl.BlockSpec((nb, lat), lambda j: (j, 0)),
            pl.BlockSpec((nb, lat), lambda j: (j, 0)),
        ],
        out_shape=[
            jax.ShapeDtypeStruct((n, lat), jnp.float32),
            jax.ShapeDtypeStruct((n, lat), jnp.float32),
        ],
    )(partials, partials, trans, resid, mu_W, mu_b.reshape(1, lat),
      lv_W, lv_b.reshape(1, lat))

    return (mu, mu, lv)


# speedup vs baseline: 22.6673x; 1.0374x over previous
import functools

import jax
import jax.numpy as jnp
from jax import lax
from jax.experimental import pallas as pl
from jax.experimental.pallas import tpu as pltpu
from jax.experimental.pallas import tpu_sc as plsc

NC = 2
NS = 16
LANES = 16
CHUNK = 80
BCH = 128
SEG = 2000
NFULL = (SEG - CHUNK) // BCH


def _make_sc_aggregate(n, h, r, e):
    ept_cnt = e // NS
    ept_agg = e // (NC * NS)
    rows_main = (n // NS) // 8 * 8
    rows_rem = n - NS * rows_main
    c_per_tile = (r * n) // NS
    c_zchunk = (c_per_tile + LANES - 1) // LANES * LANES
    seg_cnt = ept_cnt // SEG
    seg_agg = ept_agg // SEG
    assert seg_cnt * SEG == ept_cnt and NFULL * BCH + CHUNK == SEG
    assert seg_agg * SEG == ept_agg
    mesh = plsc.VectorSubcoreMesh(core_axis_name="c", subcore_axis_name="s")

    @functools.partial(
        pl.kernel,
        out_type=jax.ShapeDtypeStruct((NC * n, h), jnp.float32),
        mesh=mesh,
        compiler_params=pltpu.CompilerParams(needs_layout_passes=False),
        scratch_types=[
            pltpu.VMEM_SHARED((n, h), jnp.float32),
            pltpu.VMEM_SHARED((r * n,), jnp.float32),
            pltpu.VMEM((SEG,), jnp.int32),
            pltpu.VMEM((SEG,), jnp.int32),
            pltpu.VMEM((SEG,), jnp.int32),
            pltpu.VMEM((BCH,), jnp.float32),
            pltpu.VMEM((c_zchunk,), jnp.float32),
        ] + [pltpu.VMEM((CHUNK, h), jnp.float32),
             pltpu.VMEM((CHUNK,), jnp.int32),
             pltpu.VMEM((CHUNK,), jnp.int32),
             pltpu.VMEM((CHUNK,), jnp.int32),
             pltpu.VMEM((CHUNK,), jnp.float32)]
          + [pltpu.VMEM((BCH, h), jnp.float32),
             pltpu.VMEM((BCH,), jnp.int32),
             pltpu.VMEM((BCH,), jnp.int32),
             pltpu.VMEM((BCH,), jnp.int32),
             pltpu.VMEM((BCH,), jnp.float32)],
    )
    def sc_aggregate(etype_hbm, ei_hbm, trans_hbm,
                     out_hbm, agg_sh, c_sh, t_v, s_v, d_v, ones_v, zv_v,
                     rw_v, gx_v, rc_v, dd_v, cv_v,
                     rw2_v, gx2_v, rc2_v, dd2_v, cv2_v):
        cid = lax.axis_index("c")
        sid = lax.axis_index("s")

        def zero_rw2(i, _):
            for v in range(h // LANES):
                rw2_v[i, pl.ds(v * LANES, LANES)] = jnp.zeros(
                    (LANES,), jnp.float32)
            return ()

        lax.fori_loop(0, BCH, zero_rw2, ())
        zr = sid * rows_main
        nfull_z = rows_main // BCH
        zrem = rows_main - nfull_z * BCH
        for q in range(nfull_z):
            pltpu.sync_copy(rw2_v, agg_sh.at[pl.ds(zr + q * BCH, BCH)])
        if zrem:
            pltpu.sync_copy(rw2_v.at[pl.ds(0, zrem)],
                            agg_sh.at[pl.ds(zr + nfull_z * BCH, zrem)])

        @pl.when(sid == 0)
        def _zero_tail():
            pltpu.sync_copy(rw2_v.at[pl.ds(0, rows_rem)],
                            agg_sh.at[pl.ds(NS * rows_main, rows_rem)])

        def zero_zv(i, _):
            zv_v[pl.ds(i * LANES, LANES)] = jnp.zeros((LANES,), jnp.float32)
            return ()

        lax.fori_loop(0, c_zchunk // LANES, zero_zv, ())
        zc = sid * c_per_tile
        pltpu.sync_copy(zv_v.at[pl.ds(0, c_per_tile)],
                        c_sh.at[pl.ds(zc, c_per_tile)])
        for j in range(BCH // LANES):
            ones_v[pl.ds(j * LANES, LANES)] = jnp.ones((LANES,), jnp.float32)
        plsc.subcore_barrier()

        cnt_base = sid * ept_cnt

        def count_seg(g, _):
            e0 = cnt_base + g * SEG
            pltpu.sync_copy(etype_hbm.at[pl.ds(e0, SEG)], t_v)
            pltpu.sync_copy(ei_hbm.at[pl.ds(e + e0, SEG)], d_v)

            def count_chunk(k, _):
                for j in range(BCH // LANES):
                    sl = pl.ds(k * BCH + j * LANES, LANES)
                    ob = pl.ds(j * LANES, LANES)
                    rc2_v[ob] = t_v[sl] * n + d_v[sl]
                pltpu.sync_copy(ones_v, c_sh.at[rc2_v], add=True)
                return ()

            lax.fori_loop(0, NFULL, count_chunk, ())
            for j in range(CHUNK // LANES):
                sl = pl.ds(NFULL * BCH + j * LANES, LANES)
                ob = pl.ds(j * LANES, LANES)
                rc_v[ob] = t_v[sl] * n + d_v[sl]
            pltpu.sync_copy(ones_v.at[pl.ds(0, CHUNK)], c_sh.at[rc_v],
                            add=True)
            return ()

        lax.fori_loop(0, seg_cnt, count_seg, ())
        plsc.subcore_barrier()

        agg_base = (cid * NS + sid) * ept_agg

        def agg_seg(g, _):
            e0 = agg_base + g * SEG
            pltpu.sync_copy(etype_hbm.at[pl.ds(e0, SEG)], t_v)
            pltpu.sync_copy(ei_hbm.at[pl.ds(e0, SEG)], s_v)
            pltpu.sync_copy(ei_hbm.at[pl.ds(e + e0, SEG)], d_v)

            def do_chunk(base, sz, gxb, rcb, ddb, cvb, rwb):
                for j in range(sz // LANES):
                    sl = pl.ds(base + j * LANES, LANES)
                    ob = pl.ds(j * LANES, LANES)
                    t16 = t_v[sl]
                    d16 = d_v[sl]
                    gxb[ob] = t16 * n + s_v[sl]
                    rcb[ob] = t16 * n + d16
                    ddb[ob] = d16
                pltpu.sync_copy(trans_hbm.at[gxb], rwb)
                pltpu.sync_copy(c_sh.at[rcb], cvb)

                @plsc.parallel_loop(0, sz, 1, unroll=4)
                def scale_row(i):
                    cw = plsc.load_gather(
                        cvb, [jnp.full((LANES,), i, jnp.int32)])
                    w = 1.0 / jnp.maximum(cw, 1.0)
                    for v in range(h // LANES):
                        sl = pl.ds(v * LANES, LANES)
                        rwb[i, sl] = rwb[i, sl] * w
                pltpu.sync_copy(rwb, agg_sh.at[ddb], add=True)

            def agg_chunk(k, _):
                do_chunk(k * BCH, BCH, gx2_v, rc2_v, dd2_v, cv2_v, rw2_v)
                return ()

            lax.fori_loop(0, NFULL, agg_chunk, ())
            do_chunk(NFULL * BCH, CHUNK, gx_v, rc_v, dd_v, cv_v, rw_v)
            return ()

        lax.fori_loop(0, seg_agg, agg_seg, ())
        plsc.subcore_barrier()

        wr = sid * rows_main
        pltpu.sync_copy(agg_sh.at[pl.ds(wr, rows_main)],
                        out_hbm.at[pl.ds(cid * n + wr, rows_main)])

        @pl.when(sid == 0)
        def _write_tail():
            pltpu.sync_copy(agg_sh.at[pl.ds(NS * rows_main, rows_rem)],
                            out_hbm.at[pl.ds(cid * n + NS * rows_main,
                                             rows_rem)])

    return sc_aggregate


def _tc_trans_body(x_ref, w_ref, o_ref):
    o_ref[0] = jnp.dot(x_ref[...], w_ref[0],
                       preferred_element_type=jnp.float32)


def _tc_head_body(p0_ref, p1_ref, xw_ref, resid_ref, muw_ref, mub_ref,
                  lvw_ref, lvb_ref, mu_ref, lv_ref):
    hid = p0_ref[...] + p1_ref[...] + xw_ref[0] + resid_ref[0, 0]
    hid = jnp.maximum(hid, 0.0)
    mu_ref[...] = jnp.dot(hid, muw_ref[...],
                          preferred_element_type=jnp.float32) + mub_ref[...]
    lv_ref[...] = jnp.dot(hid, lvw_ref[...],
                          preferred_element_type=jnp.float32) + lvb_ref[...]


def kernel(edge_index, edge_type, num_nodes, node_emb, W_rel, W_self,
           mu_W, mu_b, lv_W, lv_b):
    n, h = node_emb.shape
    r = W_rel.shape[0]
    e = edge_type.shape[0]
    lat = mu_W.shape[1]
    nb = 1000
    ngrid = n // nb

    w_all = jnp.concatenate([W_rel, W_self[None]], axis=0)
    trans = pl.pallas_call(
        _tc_trans_body,
        grid=(r + 1, ngrid),
        in_specs=[
            pl.BlockSpec((nb, h), lambda i, j: (j, 0)),
            pl.BlockSpec((1, h, h), lambda i, j: (i, 0, 0)),
        ],
        out_specs=pl.BlockSpec((1, nb, h), lambda i, j: (i, j, 0)),
        out_shape=jax.ShapeDtypeStruct((r + 1, n, h), jnp.float32),
    )(node_emb, w_all)

    sc_fn = _make_sc_aggregate(n, h, r, e)
    partials = sc_fn(edge_type, edge_index.reshape(2 * e),
                     trans.reshape((r + 1) * n, h))

    resid = (jnp.asarray(num_nodes) - n).astype(jnp.float32).reshape(1, 1)
    mu, lv = pl.pallas_call(
        _tc_head_body,
        grid=(ngrid,),
        in_specs=[
            pl.BlockSpec((nb, h), lambda j: (j, 0)),
            pl.BlockSpec((nb, h), lambda j: (ngrid + j, 0)),
            pl.BlockSpec((1, nb, h), lambda j: (r, j, 0)),
            pl.BlockSpec((1, 1), lambda j: (0, 0)),
            pl.BlockSpec((h, lat), lambda j: (0, 0)),
            pl.BlockSpec((1, lat), lambda j: (0, 0)),
            pl.BlockSpec((h, lat), lambda j: (0, 0)),
            pl.BlockSpec((1, lat), lambda j: (0, 0)),
        ],
        out_specs=[
            pl.BlockSpec((nb, lat), lambda j: (j, 0)),
            pl.BlockSpec((nb, lat), lambda j: (j, 0)),
        ],
        out_shape=[
            jax.ShapeDtypeStruct((n, lat), jnp.float32),
            jax.ShapeDtypeStruct((n, lat), jnp.float32),
        ],
    )(partials, partials, trans, resid, mu_W, mu_b.reshape(1, lat),
      lv_W, lv_b.reshape(1, lat))

    return (mu, mu, lv)


# speedup vs baseline: 24.6948x; 1.0894x over previous
import functools

import jax
import jax.numpy as jnp
from jax import lax
from jax.experimental import pallas as pl
from jax.experimental.pallas import tpu as pltpu
from jax.experimental.pallas import tpu_sc as plsc

NC = 2
NS = 16
LANES = 16
CHUNK = 80
BCH = 128
SEG = 2000
NFULL = (SEG - CHUNK) // BCH


def _make_sc_count(n, r, e):
    ept = e // (NC * NS)
    c_per_tile = (r * n) // NS
    c_zchunk = (c_per_tile + LANES - 1) // LANES * LANES
    seg_cnt = ept // SEG
    assert seg_cnt * SEG == ept
    mesh = plsc.VectorSubcoreMesh(core_axis_name="c", subcore_axis_name="s")

    @functools.partial(
        pl.kernel,
        out_type=jax.ShapeDtypeStruct((NC * r * n,), jnp.float32),
        mesh=mesh,
        compiler_params=pltpu.CompilerParams(needs_layout_passes=False),
        scratch_types=[
            pltpu.VMEM_SHARED((r * n,), jnp.float32),
            pltpu.VMEM((SEG,), jnp.int32),
            pltpu.VMEM((SEG,), jnp.int32),
            pltpu.VMEM((BCH,), jnp.float32),
            pltpu.VMEM((c_zchunk,), jnp.float32),
            pltpu.VMEM((BCH,), jnp.int32),
            pltpu.VMEM((CHUNK,), jnp.int32),
        ],
    )
    def sc_count(etype_hbm, ei_hbm, out_hbm, c_sh, t_v, d_v, ones_v,
                 zv_v, rc2_v, rc_v):
        cid = lax.axis_index("c")
        sid = lax.axis_index("s")

        def zero_zv(i, _):
            zv_v[pl.ds(i * LANES, LANES)] = jnp.zeros((LANES,), jnp.float32)
            return ()

        lax.fori_loop(0, c_zchunk // LANES, zero_zv, ())
        zc = sid * c_per_tile
        pltpu.sync_copy(zv_v.at[pl.ds(0, c_per_tile)],
                        c_sh.at[pl.ds(zc, c_per_tile)])
        for j in range(BCH // LANES):
            ones_v[pl.ds(j * LANES, LANES)] = jnp.ones((LANES,), jnp.float32)
        plsc.subcore_barrier()

        cnt_base = (cid * NS + sid) * ept

        def count_seg(g, _):
            e0 = cnt_base + g * SEG
            pltpu.sync_copy(etype_hbm.at[pl.ds(e0, SEG)], t_v)
            pltpu.sync_copy(ei_hbm.at[pl.ds(e + e0, SEG)], d_v)

            def count_chunk(k, _):
                for j in range(BCH // LANES):
                    sl = pl.ds(k * BCH + j * LANES, LANES)
                    ob = pl.ds(j * LANES, LANES)
                    rc2_v[ob] = t_v[sl] * n + d_v[sl]
                pltpu.sync_copy(ones_v, c_sh.at[rc2_v], add=True)
                return ()

            lax.fori_loop(0, NFULL, count_chunk, ())
            for j in range(CHUNK // LANES):
                sl = pl.ds(NFULL * BCH + j * LANES, LANES)
                ob = pl.ds(j * LANES, LANES)
                rc_v[ob] = t_v[sl] * n + d_v[sl]
            pltpu.sync_copy(ones_v.at[pl.ds(0, CHUNK)], c_sh.at[rc_v],
                            add=True)
            return ()

        lax.fori_loop(0, seg_cnt, count_seg, ())
        plsc.subcore_barrier()
        wc = sid * c_per_tile
        pltpu.sync_copy(c_sh.at[pl.ds(wc, c_per_tile)],
                        zv_v.at[pl.ds(0, c_per_tile)])
        pltpu.sync_copy(zv_v.at[pl.ds(0, c_per_tile)],
                        out_hbm.at[pl.ds(cid * r * n + wc, c_per_tile)])

    return sc_count


def _make_sc_aggregate(n, h, r, e):
    ept_cnt = e // NS
    ept_agg = e // (NC * NS)
    rows_main = (n // NS) // 8 * 8
    rows_rem = n - NS * rows_main
    c_per_tile = (r * n) // NS
    c_zchunk = (c_per_tile + LANES - 1) // LANES * LANES
    seg_agg = ept_agg // SEG
    assert NFULL * BCH + CHUNK == SEG
    assert seg_agg * SEG == ept_agg
    mesh = plsc.VectorSubcoreMesh(core_axis_name="c", subcore_axis_name="s")

    @functools.partial(
        pl.kernel,
        out_type=jax.ShapeDtypeStruct((NC * n, h), jnp.float32),
        mesh=mesh,
        compiler_params=pltpu.CompilerParams(needs_layout_passes=False),
        scratch_types=[
            pltpu.VMEM_SHARED((n, h), jnp.float32),
            pltpu.VMEM_SHARED((r * n,), jnp.float32),
            pltpu.VMEM((SEG,), jnp.int32),
            pltpu.VMEM((SEG,), jnp.int32),
            pltpu.VMEM((SEG,), jnp.int32),
            pltpu.VMEM((BCH,), jnp.float32),
            pltpu.VMEM((c_zchunk,), jnp.float32),
            pltpu.VMEM((c_zchunk,), jnp.float32),
        ] + [pltpu.VMEM((CHUNK, h), jnp.float32),
             pltpu.VMEM((CHUNK,), jnp.int32),
             pltpu.VMEM((CHUNK,), jnp.int32),
             pltpu.VMEM((CHUNK,), jnp.int32),
             pltpu.VMEM((CHUNK,), jnp.float32)]
          + [pltpu.VMEM((BCH, h), jnp.float32),
             pltpu.VMEM((BCH,), jnp.int32),
             pltpu.VMEM((BCH,), jnp.int32),
             pltpu.VMEM((BCH,), jnp.int32),
             pltpu.VMEM((BCH,), jnp.float32)],
    )
    def sc_aggregate(etype_hbm, ei_hbm, trans_hbm, cpart_hbm,
                     out_hbm, agg_sh, c_sh, t_v, s_v, d_v, ones_v, zv_v,
                     cz_v, rw_v, gx_v, rc_v, dd_v, cv_v,
                     rw2_v, gx2_v, rc2_v, dd2_v, cv2_v):
        cid = lax.axis_index("c")
        sid = lax.axis_index("s")

        def zero_rw2(i, _):
            for v in range(h // LANES):
                rw2_v[i, pl.ds(v * LANES, LANES)] = jnp.zeros(
                    (LANES,), jnp.float32)
            return ()

        lax.fori_loop(0, BCH, zero_rw2, ())
        zr = sid * rows_main
        nfull_z = rows_main // BCH
        zrem = rows_main - nfull_z * BCH
        for q in range(nfull_z):
            pltpu.sync_copy(rw2_v, agg_sh.at[pl.ds(zr + q * BCH, BCH)])
        if zrem:
            pltpu.sync_copy(rw2_v.at[pl.ds(0, zrem)],
                            agg_sh.at[pl.ds(zr + nfull_z * BCH, zrem)])

        @pl.when(sid == 0)
        def _zero_tail():
            pltpu.sync_copy(rw2_v.at[pl.ds(0, rows_rem)],
                            agg_sh.at[pl.ds(NS * rows_main, rows_rem)])

        def zero_zv(i, _):
            zv_v[pl.ds(i * LANES, LANES)] = jnp.zeros((LANES,), jnp.float32)
            return ()

        zc = sid * c_per_tile
        pltpu.sync_copy(cpart_hbm.at[pl.ds(zc, c_per_tile)],
                        zv_v.at[pl.ds(0, c_per_tile)])
        pltpu.sync_copy(cpart_hbm.at[pl.ds(r * n + zc, c_per_tile)],
                        cz_v.at[pl.ds(0, c_per_tile)])

        def sum_counts(i, _):
            sl = pl.ds(i * LANES, LANES)
            zv_v[sl] = zv_v[sl] + cz_v[sl]
            return ()

        lax.fori_loop(0, c_zchunk // LANES, sum_counts, ())
        pltpu.sync_copy(zv_v.at[pl.ds(0, c_per_tile)],
                        c_sh.at[pl.ds(zc, c_per_tile)])
        plsc.subcore_barrier()

        agg_base = (cid * NS + sid) * ept_agg

        def agg_seg(g, _):
            e0 = agg_base + g * SEG
            pltpu.sync_copy(etype_hbm.at[pl.ds(e0, SEG)], t_v)
            pltpu.sync_copy(ei_hbm.at[pl.ds(e0, SEG)], s_v)
            pltpu.sync_copy(ei_hbm.at[pl.ds(e + e0, SEG)], d_v)

            def do_chunk(base, sz, gxb, rcb, ddb, cvb, rwb):
                for j in range(sz // LANES):
                    sl = pl.ds(base + j * LANES, LANES)
                    ob = pl.ds(j * LANES, LANES)
                    t16 = t_v[sl]
                    d16 = d_v[sl]
                    gxb[ob] = t16 * n + s_v[sl]
                    rcb[ob] = t16 * n + d16
                    ddb[ob] = d16
                pltpu.sync_copy(trans_hbm.at[gxb], rwb)
                pltpu.sync_copy(c_sh.at[rcb], cvb)

                @plsc.parallel_loop(0, sz, 1, unroll=4)
                def scale_row(i):
                    cw = plsc.load_gather(
                        cvb, [jnp.full((LANES,), i, jnp.int32)])
                    w = 1.0 / jnp.maximum(cw, 1.0)
                    for v in range(h // LANES):
                        sl = pl.ds(v * LANES, LANES)
                        rwb[i, sl] = rwb[i, sl] * w
                pltpu.sync_copy(rwb, agg_sh.at[ddb], add=True)

            def agg_chunk(k, _):
                do_chunk(k * BCH, BCH, gx2_v, rc2_v, dd2_v, cv2_v, rw2_v)
                return ()

            lax.fori_loop(0, NFULL, agg_chunk, ())
            do_chunk(NFULL * BCH, CHUNK, gx_v, rc_v, dd_v, cv_v, rw_v)
            return ()

        lax.fori_loop(0, seg_agg, agg_seg, ())
        plsc.subcore_barrier()

        wr = sid * rows_main
        pltpu.sync_copy(agg_sh.at[pl.ds(wr, rows_main)],
                        out_hbm.at[pl.ds(cid * n + wr, rows_main)])

        @pl.when(sid == 0)
        def _write_tail():
            pltpu.sync_copy(agg_sh.at[pl.ds(NS * rows_main, rows_rem)],
                            out_hbm.at[pl.ds(cid * n + NS * rows_main,
                                             rows_rem)])

    return sc_aggregate


def _tc_trans_body(x_ref, w_ref, o_ref):
    o_ref[0] = jnp.dot(x_ref[...], w_ref[0],
                       preferred_element_type=jnp.float32)


def _tc_head_body(p0_ref, p1_ref, xw_ref, resid_ref, muw_ref, mub_ref,
                  lvw_ref, lvb_ref, mu_ref, lv_ref):
    hid = p0_ref[...] + p1_ref[...] + xw_ref[0] + resid_ref[0, 0]
    hid = jnp.maximum(hid, 0.0)
    mu_ref[...] = jnp.dot(hid, muw_ref[...],
                          preferred_element_type=jnp.float32) + mub_ref[...]
    lv_ref[...] = jnp.dot(hid, lvw_ref[...],
                          preferred_element_type=jnp.float32) + lvb_ref[...]


def kernel(edge_index, edge_type, num_nodes, node_emb, W_rel, W_self,
           mu_W, mu_b, lv_W, lv_b):
    n, h = node_emb.shape
    r = W_rel.shape[0]
    e = edge_type.shape[0]
    lat = mu_W.shape[1]
    nb = 1000
    ngrid = n // nb

    w_all = jnp.concatenate([W_rel, W_self[None]], axis=0)
    trans = pl.pallas_call(
        _tc_trans_body,
        grid=(r + 1, ngrid),
        in_specs=[
            pl.BlockSpec((nb, h), lambda i, j: (j, 0)),
            pl.BlockSpec((1, h, h), lambda i, j: (i, 0, 0)),
        ],
        out_specs=pl.BlockSpec((1, nb, h), lambda i, j: (i, j, 0)),
        out_shape=jax.ShapeDtypeStruct((r + 1, n, h), jnp.float32),
    )(node_emb, w_all)

    ei_flat = edge_index.reshape(2 * e)
    cpart = _make_sc_count(n, r, e)(edge_type, ei_flat)
    sc_fn = _make_sc_aggregate(n, h, r, e)
    partials = sc_fn(edge_type, ei_flat, trans.reshape((r + 1) * n, h),
                     cpart)

    resid = (jnp.asarray(num_nodes) - n).astype(jnp.float32).reshape(1, 1)
    mu, lv = pl.pallas_call(
        _tc_head_body,
        grid=(ngrid,),
        in_specs=[
            pl.BlockSpec((nb, h), lambda j: (j, 0)),
            pl.BlockSpec((nb, h), lambda j: (ngrid + j, 0)),
            pl.BlockSpec((1, nb, h), lambda j: (r, j, 0)),
            pl.BlockSpec((1, 1), lambda j: (0, 0)),
            pl.BlockSpec((h, lat), lambda j: (0, 0)),
            pl.BlockSpec((1, lat), lambda j: (0, 0)),
            pl.BlockSpec((h, lat), lambda j: (0, 0)),
            pl.BlockSpec((1, lat), lambda j: (0, 0)),
        ],
        out_specs=[
            pl.BlockSpec((nb, lat), lambda j: (j, 0)),
            pl.BlockSpec((nb, lat), lambda j: (j, 0)),
        ],
        out_shape=[
            jax.ShapeDtypeStruct((n, lat), jnp.float32),
            jax.ShapeDtypeStruct((n, lat), jnp.float32),
        ],
    )(partials, partials, trans, resid, mu_W, mu_b.reshape(1, lat),
      lv_W, lv_b.reshape(1, lat))

    return (mu, mu, lv)
